# Initial kernel scaffold; baseline (speedup 1.0000x reference)
import jax
import jax.numpy as jnp
from jax.experimental import pallas as pl


def kernel(obs, instr, w256, w32):
    raise NotImplementedError("write your pallas kernel here")


# speedup vs baseline: 4.4094x; 4.4094x over previous
import functools

import jax
import jax.numpy as jnp
from jax.experimental import pallas as pl
from jax.experimental.pallas import tpu as pltpu

_VX = 8
_SFC = 256
_ED = 32
_TBL = 80
_BB = 4096


def _table_kernel(w256_ref, w32_ref, tbl_ref):
    f32 = jnp.float32
    w32 = w32_ref[...]
    row = jax.lax.broadcasted_iota(jnp.int32, (64, 17), 0)
    col = jax.lax.broadcasted_iota(jnp.int32, (64, 17), 1)
    oh = jnp.logical_or(jnp.logical_or(col == row // _VX,
                                       col == _VX + row % _VX),
                        col == 16).astype(f32)
    h = jnp.maximum(jnp.dot(oh, w256_ref[...], preferred_element_type=f32), 0.0)
    se = (jnp.dot(h, w32[:_SFC, :], preferred_element_type=f32)
          + w32[_SFC + _ED + 10:_SFC + _ED + 11, :])
    a = jnp.dot(jnp.maximum(se, 0.0), w32[_SFC:_SFC + _ED, :],
                preferred_element_type=f32)
    t_i = w32[_SFC + _ED:_SFC + _ED + 10, :]
    tbl_ref[...] = jnp.concatenate(
        [a, t_i, jnp.zeros((_TBL - 74, _ED), f32)], axis=0)


def _lookup_kernel(obs_ref, instr_ref, tbl_ref, out_ref):
    f32 = jnp.float32
    obs = obs_ref[...]
    j_xy = obs[:, 0:1] * _VX + obs[:, 1:2]
    j_in = instr_ref[...] + 64
    lane = jax.lax.broadcasted_iota(jnp.int32, (_BB, _TBL), 1)
    tgt = jnp.where(lane < 64, j_xy, j_in)
    oh = (lane == tgt).astype(f32)
    out_ref[...] = jnp.dot(oh, tbl_ref[...], preferred_element_type=f32)


@jax.jit
def kernel(obs, instr, w256, w32):
    n = obs.shape[0]
    tbl = pl.pallas_call(
        _table_kernel,
        out_shape=jax.ShapeDtypeStruct((_TBL, _ED), jnp.float32),
    )(w256, w32)

    grid = (n // _BB,)
    return pl.pallas_call(
        _lookup_kernel,
        out_shape=jax.ShapeDtypeStruct((n, _ED), jnp.float32),
        grid=grid,
        in_specs=[pl.BlockSpec((_BB, 2), lambda i: (i, 0)),
                  pl.BlockSpec((_BB, 1), lambda i: (i, 0)),
                  pl.BlockSpec((_TBL, _ED), lambda i: (0, 0))],
        out_specs=pl.BlockSpec((_BB, _ED), lambda i: (i, 0)),
        compiler_params=pltpu.CompilerParams(
            dimension_semantics=("parallel",)),
    )(obs, instr, tbl)


# speedup vs baseline: 20.4042x; 4.6274x over previous
import jax
import jax.numpy as jnp
from jax.experimental import pallas as pl
from jax.experimental.pallas import tpu as pltpu

_VX = 8
_SFC = 256
_ED = 32
_TBL = 80
_L = 2048


def _table_kernel(w256_ref, w32_ref, tbl_ref):
    f32 = jnp.float32
    w32 = w32_ref[...]
    row = jax.lax.broadcasted_iota(jnp.int32, (64, 17), 0)
    col = jax.lax.broadcasted_iota(jnp.int32, (64, 17), 1)
    oh = jnp.logical_or(jnp.logical_or(col == row // _VX,
                                       col == _VX + row % _VX),
                        col == 16).astype(f32)
    h = jnp.maximum(jnp.dot(oh, w256_ref[...], preferred_element_type=f32), 0.0)
    se = (jnp.dot(h, w32[:_SFC, :], preferred_element_type=f32)
          + w32[_SFC + _ED + 10:_SFC + _ED + 11, :])
    a = jnp.dot(jnp.maximum(se, 0.0), w32[_SFC:_SFC + _ED, :],
                preferred_element_type=f32)
    t_i = w32[_SFC + _ED:_SFC + _ED + 10, :]
    tbl = jnp.concatenate(
        [a, t_i, jnp.zeros((_TBL - 74, _ED), f32)], axis=0)
    tbl_ref[...] = tbl.T


def _lookup_kernel(obs_ref, instr_ref, tbl_ref, out_ref):
    f32 = jnp.float32
    obs = obs_ref[...]
    j_xy = obs[0:1, :] * _VX + obs[1:2, :]
    j_in = instr_ref[...] + 64
    sub = jax.lax.broadcasted_iota(jnp.int32, (_TBL, _L), 0)
    tgt = jnp.where(sub < 64, j_xy, j_in)
    oh = (sub == tgt).astype(f32)
    out_ref[...] = jnp.dot(tbl_ref[...], oh, preferred_element_type=f32)


@jax.jit
def kernel(obs, instr, w256, w32):
    n = obs.shape[0]
    tbl_t = pl.pallas_call(
        _table_kernel,
        out_shape=jax.ShapeDtypeStruct((_ED, _TBL), jnp.float32),
    )(w256, w32)

    obs_t = obs.T
    instr_t = instr.T
    grid = (n // _L,)
    out_t = pl.pallas_call(
        _lookup_kernel,
        out_shape=jax.ShapeDtypeStruct((_ED, n), jnp.float32),
        grid=grid,
        in_specs=[pl.BlockSpec((2, _L), lambda i: (0, i)),
                  pl.BlockSpec((1, _L), lambda i: (0, i)),
                  pl.BlockSpec((_ED, _TBL), lambda i: (0, 0))],
        out_specs=pl.BlockSpec((_ED, _L), lambda i: (0, i)),
        compiler_params=pltpu.CompilerParams(
            dimension_semantics=("parallel",)),
    )(obs_t, instr_t, tbl_t)
    return out_t.T


# speedup vs baseline: 57.9777x; 2.8415x over previous
import jax
import jax.numpy as jnp
from jax.experimental import pallas as pl
from jax.experimental.pallas import tpu as pltpu

_VX = 8
_SFC = 256
_ED = 32
_TBL = 80
_L = 8192


def _table_kernel(w256_ref, w32_ref, tbl_ref):
    f32 = jnp.float32
    w32 = w32_ref[...]
    row = jax.lax.broadcasted_iota(jnp.int32, (64, 17), 0)
    col = jax.lax.broadcasted_iota(jnp.int32, (64, 17), 1)
    oh = jnp.logical_or(jnp.logical_or(col == row // _VX,
                                       col == _VX + row % _VX),
                        col == 16).astype(f32)
    h = jnp.maximum(jnp.dot(oh, w256_ref[...], preferred_element_type=f32), 0.0)
    se = (jnp.dot(h, w32[:_SFC, :], preferred_element_type=f32)
          + w32[_SFC + _ED + 10:_SFC + _ED + 11, :])
    a = jnp.dot(jnp.maximum(se, 0.0), w32[_SFC:_SFC + _ED, :],
                preferred_element_type=f32)
    t_i = w32[_SFC + _ED:_SFC + _ED + 10, :]
    tbl = jnp.concatenate(
        [a, t_i, jnp.zeros((_TBL - 74, _ED), f32)], axis=0)
    tbl_ref[...] = tbl.T


def _lookup_kernel(obs_ref, instr_ref, tbl_ref, out_ref):
    f32 = jnp.float32
    obs = obs_ref[...]
    j_xy = obs[0:1, :] * _VX + obs[1:2, :]
    j_in = instr_ref[...] + 64
    sub = jax.lax.broadcasted_iota(jnp.int32, (_TBL, _L), 0)
    tgt = jnp.where(sub < 64, j_xy, j_in)
    oh = (sub == tgt).astype(f32)
    out_ref[...] = jnp.dot(tbl_ref[...], oh, preferred_element_type=f32)


@jax.jit
def kernel(obs, instr, w256, w32):
    n = obs.shape[0]
    tbl_t = pl.pallas_call(
        _table_kernel,
        out_shape=jax.ShapeDtypeStruct((_ED, _TBL), jnp.float32),
    )(w256, w32)

    obs_t = obs.T
    instr_t = instr.T
    grid = (n // _L,)
    out_t = pl.pallas_call(
        _lookup_kernel,
        out_shape=jax.ShapeDtypeStruct((_ED, n), jnp.float32),
        grid=grid,
        in_specs=[pl.BlockSpec((2, _L), lambda i: (0, i)),
                  pl.BlockSpec((1, _L), lambda i: (0, i)),
                  pl.BlockSpec((_ED, _TBL), lambda i: (0, 0))],
        out_specs=pl.BlockSpec((_ED, _L), lambda i: (0, i)),
        compiler_params=pltpu.CompilerParams(
            dimension_semantics=("parallel",)),
    )(obs_t, instr_t, tbl_t)
    return out_t.T


# speedup vs baseline: 86.1267x; 1.4855x over previous
import jax
import jax.numpy as jnp
from jax.experimental import pallas as pl
from jax.experimental.pallas import tpu as pltpu

_VX = 8
_SFC = 256
_ED = 32
_TBL = 80
_L = 16384


def _table_kernel(w256_ref, w32_ref, tbl_ref):
    f32 = jnp.float32
    w32 = w32_ref[...]
    row = jax.lax.broadcasted_iota(jnp.int32, (64, 17), 0)
    col = jax.lax.broadcasted_iota(jnp.int32, (64, 17), 1)
    oh = jnp.logical_or(jnp.logical_or(col == row // _VX,
                                       col == _VX + row % _VX),
                        col == 16).astype(f32)
    h = jnp.maximum(jnp.dot(oh, w256_ref[...], preferred_element_type=f32), 0.0)
    se = (jnp.dot(h, w32[:_SFC, :], preferred_element_type=f32)
          + w32[_SFC + _ED + 10:_SFC + _ED + 11, :])
    a = jnp.dot(jnp.maximum(se, 0.0), w32[_SFC:_SFC + _ED, :],
                preferred_element_type=f32)
    t_i = w32[_SFC + _ED:_SFC + _ED + 10, :]
    tbl = jnp.concatenate(
        [a, t_i, jnp.zeros((_TBL - 74, _ED), f32)], axis=0)
    tbl_ref[...] = tbl.T


def _lookup_kernel(obs_ref, instr_ref, tbl_ref, out_ref):
    f32 = jnp.float32
    obs = obs_ref[...]
    j_xy = obs[0:1, :] * _VX + obs[1:2, :]
    j_in = instr_ref[...] + 64
    sub = jax.lax.broadcasted_iota(jnp.int32, (_TBL, _L), 0)
    tgt = jnp.where(sub < 64, j_xy, j_in)
    oh = (sub == tgt).astype(f32)
    out_ref[...] = jnp.dot(tbl_ref[...], oh, preferred_element_type=f32)


@jax.jit
def kernel(obs, instr, w256, w32):
    n = obs.shape[0]
    tbl_t = pl.pallas_call(
        _table_kernel,
        out_shape=jax.ShapeDtypeStruct((_ED, _TBL), jnp.float32),
    )(w256, w32)

    obs_t = obs.T
    instr_t = instr.T
    grid = (n // _L,)
    out_t = pl.pallas_call(
        _lookup_kernel,
        out_shape=jax.ShapeDtypeStruct((_ED, n), jnp.float32),
        grid=grid,
        in_specs=[pl.BlockSpec((2, _L), lambda i: (0, i)),
                  pl.BlockSpec((1, _L), lambda i: (0, i)),
                  pl.BlockSpec((_ED, _TBL), lambda i: (0, 0))],
        out_specs=pl.BlockSpec((_ED, _L), lambda i: (0, i)),
        compiler_params=pltpu.CompilerParams(
            dimension_semantics=("parallel",)),
    )(obs_t, instr_t, tbl_t)
    return out_t.T


# speedup vs baseline: 112.0069x; 1.3005x over previous
import jax
import jax.numpy as jnp
from jax.experimental import pallas as pl
from jax.experimental.pallas import tpu as pltpu

_VX = 8
_SFC = 256
_ED = 32
_TBL = 80
_L = 32768


def _table_kernel(w256_ref, w32_ref, tbl_ref):
    f32 = jnp.float32
    w32 = w32_ref[...]
    row = jax.lax.broadcasted_iota(jnp.int32, (64, 17), 0)
    col = jax.lax.broadcasted_iota(jnp.int32, (64, 17), 1)
    oh = jnp.logical_or(jnp.logical_or(col == row // _VX,
                                       col == _VX + row % _VX),
                        col == 16).astype(f32)
    h = jnp.maximum(jnp.dot(oh, w256_ref[...], preferred_element_type=f32), 0.0)
    se = (jnp.dot(h, w32[:_SFC, :], preferred_element_type=f32)
          + w32[_SFC + _ED + 10:_SFC + _ED + 11, :])
    a = jnp.dot(jnp.maximum(se, 0.0), w32[_SFC:_SFC + _ED, :],
                preferred_element_type=f32)
    t_i = w32[_SFC + _ED:_SFC + _ED + 10, :]
    tbl = jnp.concatenate(
        [a, t_i, jnp.zeros((_TBL - 74, _ED), f32)], axis=0)
    tbl_ref[...] = tbl.T


def _lookup_kernel(obs_ref, instr_ref, tbl_ref, out_ref):
    f32 = jnp.float32
    obs = obs_ref[...]
    j_xy = obs[0:1, :] * _VX + obs[1:2, :]
    j_in = instr_ref[...] + 64
    sub = jax.lax.broadcasted_iota(jnp.int32, (_TBL, _L), 0)
    tgt = jnp.where(sub < 64, j_xy, j_in)
    oh = (sub == tgt).astype(f32)
    out_ref[...] = jnp.dot(tbl_ref[...], oh, preferred_element_type=f32)


@jax.jit
def kernel(obs, instr, w256, w32):
    n = obs.shape[0]
    tbl_t = pl.pallas_call(
        _table_kernel,
        out_shape=jax.ShapeDtypeStruct((_ED, _TBL), jnp.float32),
    )(w256, w32)

    obs_t = obs.T
    instr_t = instr.T
    grid = (n // _L,)
    out_t = pl.pallas_call(
        _lookup_kernel,
        out_shape=jax.ShapeDtypeStruct((_ED, n), jnp.float32),
        grid=grid,
        in_specs=[pl.BlockSpec((2, _L), lambda i: (0, i)),
                  pl.BlockSpec((1, _L), lambda i: (0, i)),
                  pl.BlockSpec((_ED, _TBL), lambda i: (0, 0))],
        out_specs=pl.BlockSpec((_ED, _L), lambda i: (0, i)),
        compiler_params=pltpu.CompilerParams(
            dimension_semantics=("parallel",)),
    )(obs_t, instr_t, tbl_t)
    return out_t.T


# speedup vs baseline: 128.9578x; 1.1513x over previous
import jax
import jax.numpy as jnp
from jax.experimental import pallas as pl
from jax.experimental.pallas import tpu as pltpu

_VX = 8
_SFC = 256
_ED = 32
_TBL = 80
_L = 65536


def _table_kernel(w256_ref, w32_ref, tbl_ref):
    f32 = jnp.float32
    w32 = w32_ref[...]
    row = jax.lax.broadcasted_iota(jnp.int32, (64, 17), 0)
    col = jax.lax.broadcasted_iota(jnp.int32, (64, 17), 1)
    oh = jnp.logical_or(jnp.logical_or(col == row // _VX,
                                       col == _VX + row % _VX),
                        col == 16).astype(f32)
    h = jnp.maximum(jnp.dot(oh, w256_ref[...], preferred_element_type=f32), 0.0)
    se = (jnp.dot(h, w32[:_SFC, :], preferred_element_type=f32)
          + w32[_SFC + _ED + 10:_SFC + _ED + 11, :])
    a = jnp.dot(jnp.maximum(se, 0.0), w32[_SFC:_SFC + _ED, :],
                preferred_element_type=f32)
    t_i = w32[_SFC + _ED:_SFC + _ED + 10, :]
    tbl = jnp.concatenate(
        [a, t_i, jnp.zeros((_TBL - 74, _ED), f32)], axis=0)
    tbl_ref[...] = tbl.T


def _lookup_kernel(obs_ref, instr_ref, tbl_ref, out_ref):
    f32 = jnp.float32
    obs = obs_ref[...]
    j_xy = obs[0:1, :] * _VX + obs[1:2, :]
    j_in = instr_ref[...] + 64
    sub = jax.lax.broadcasted_iota(jnp.int32, (_TBL, _L), 0)
    tgt = jnp.where(sub < 64, j_xy, j_in)
    oh = (sub == tgt).astype(f32)
    out_ref[...] = jnp.dot(tbl_ref[...], oh, preferred_element_type=f32)


@jax.jit
def kernel(obs, instr, w256, w32):
    n = obs.shape[0]
    tbl_t = pl.pallas_call(
        _table_kernel,
        out_shape=jax.ShapeDtypeStruct((_ED, _TBL), jnp.float32),
    )(w256, w32)

    obs_t = obs.T
    instr_t = instr.T
    grid = (n // _L,)
    out_t = pl.pallas_call(
        _lookup_kernel,
        out_shape=jax.ShapeDtypeStruct((_ED, n), jnp.float32),
        grid=grid,
        in_specs=[pl.BlockSpec((2, _L), lambda i: (0, i)),
                  pl.BlockSpec((1, _L), lambda i: (0, i)),
                  pl.BlockSpec((_ED, _TBL), lambda i: (0, 0))],
        out_specs=pl.BlockSpec((_ED, _L), lambda i: (0, i)),
        compiler_params=pltpu.CompilerParams(
            dimension_semantics=("parallel",)),
    )(obs_t, instr_t, tbl_t)
    return out_t.T
